# baseline (speedup 1.0000x reference)
import functools

import jax
import jax.numpy as jnp
from jax import lax
from jax.experimental import pallas as pl
from jax.experimental.pallas import tpu as pltpu
from jax.experimental.pallas import tpu_sc as plsc

_B, _T, _D = 4, 2048, 128
_NC, _NS, _L = 2, 16, 16
_NW = _NC * _NS
_R = _T // _NW


def _sc_batch0(x, pos_table):
    mesh = plsc.VectorSubcoreMesh(core_axis_name="c", subcore_axis_name="s")

    @functools.partial(
        pl.kernel,
        out_type=jax.ShapeDtypeStruct((_B, _T, _D), jnp.float32),
        mesh=mesh,
        scratch_types=[
            pltpu.VMEM((_R, _D), jnp.float32),
            pltpu.VMEM((_R, _D), jnp.float32),
            pltpu.SemaphoreType.DMA,
            pltpu.SemaphoreType.DMA,
        ],
    )
    def k(x_hbm, pos_hbm, out_hbm, pos_v, buf, si, so):
        wid = lax.axis_index("s") * _NC + lax.axis_index("c")
        row0 = wid * _R
        pload = pltpu.async_copy(pos_hbm.at[pl.ds(row0, _R), :], pos_v, si)
        xload = pltpu.async_copy(x_hbm.at[0, pl.ds(row0, _R), :], buf, si)
        pload.wait()
        xload.wait()

        def body(r):
            for c in range(0, _D, _L):
                plsc.addupdate(buf.at[pl.ds(r, 1), pl.ds(c, _L)],
                               pos_v.at[pl.ds(r, 1), pl.ds(c, _L)][...])

        plsc.parallel_loop(0, _R, 1, unroll=4)(body)
        pltpu.async_copy(buf, out_hbm.at[0, pl.ds(row0, _R), :], so).wait()

    return k(x, pos_table)


def _tc_body(y_ref, x_ref, p_ref, o_ref):
    o_ref[...] = x_ref[...] + p_ref[...]


def _tc_rest(y, x, pos_table):
    return pl.pallas_call(
        _tc_body,
        grid=(_B - 1,),
        in_specs=[
            pl.BlockSpec((1, 8, _D), lambda b: (0, 0, 0)),
            pl.BlockSpec((1, _T, _D), lambda b: (b + 1, 0, 0)),
            pl.BlockSpec((_T, _D), lambda b: (0, 0)),
        ],
        out_specs=pl.BlockSpec((1, _T, _D), lambda b: (b + 1, 0, 0)),
        out_shape=jax.ShapeDtypeStruct((_B, _T, _D), jnp.float32),
        input_output_aliases={0: 0},
    )(y, x, pos_table)


@jax.jit
def _split(x, pos_table):
    y = _sc_batch0(x, pos_table)
    return _tc_rest(y, x, pos_table)


def kernel(x, pos_table):
    return _split(x, pos_table)


# speedup vs baseline: 1.0820x; 1.0820x over previous
import functools

import jax
import jax.numpy as jnp
from jax import lax
from jax.experimental import pallas as pl
from jax.experimental.pallas import tpu as pltpu
from jax.experimental.pallas import tpu_sc as plsc

_B, _T, _D = 4, 2048, 128
_NC, _NS, _L = 2, 16, 16
_NW = _NC * _NS
_R = _T // _NW


@jax.jit
def _sc_add(x, pos_table):
    mesh = plsc.VectorSubcoreMesh(core_axis_name="c", subcore_axis_name="s")

    @functools.partial(
        pl.kernel,
        out_type=jax.ShapeDtypeStruct((_B, _T, _D), jnp.float32),
        mesh=mesh,
        scratch_types=[
            pltpu.VMEM((_R, _D), jnp.float32),
            pltpu.VMEM((_B, _R, _D), jnp.float32),
            pltpu.SemaphoreType.DMA,
            pltpu.SemaphoreType.DMA,
        ],
    )
    def k(x_hbm, pos_hbm, out_hbm, pos_v, bufs, si, so):
        wid = lax.axis_index("s") * _NC + lax.axis_index("c")
        row0 = wid * _R
        pload = pltpu.async_copy(pos_hbm.at[pl.ds(row0, _R), :], pos_v, si)
        loads = []
        for b in range(_B):
            loads.append(
                pltpu.async_copy(x_hbm.at[b, pl.ds(row0, _R), :],
                                 bufs.at[b], si))
        pload.wait()
        for b in range(_B):
            loads[b].wait()
        stores = []
        hr = _R // 2
        for h in range(2):
            def body(r):
                for c in range(0, _D, _L):
                    p = pos_v.at[pl.ds(r, 1), pl.ds(c, _L)][...]
                    for b in range(_B):
                        plsc.addupdate(
                            bufs.at[b].at[pl.ds(r, 1), pl.ds(c, _L)], p)

            plsc.parallel_loop(h * hr, (h + 1) * hr, 1, unroll=2)(body)
            for b in range(_B):
                stores.append(
                    pltpu.async_copy(
                        bufs.at[b].at[pl.ds(h * hr, hr)],
                        out_hbm.at[b, pl.ds(row0 + h * hr, hr), :], so))
        for st in stores:
            st.wait()

    return k(x, pos_table)


def kernel(x, pos_table):
    return _sc_add(x, pos_table)
